# baseline (speedup 1.0000x reference)
import functools

import jax
import jax.numpy as jnp
from jax import lax
from jax.experimental import pallas as pl
from jax.experimental.pallas import tpu as pltpu
from jax.experimental.pallas import tpu_sc as plsc

N = 100000
NPAD = 100096
D = 16
NZ = 64
TROWS = 2 * N + NZ
NK = 131072

NW = 32
NP = 8
KSLICE = NK // NP
NODE_SPAN = 3200
NODE_STRIDE = 3120
NCHUNK = 1600
IDXROWS = 13
CSPAN = IDXROWS * 128

_SC_PARAMS = pltpu.CompilerParams(needs_layout_passes=False,
                                  use_tc_tiling_on_sc=False)


def _table_body(x8_ref, f8_ref, w1x_ref, w1f_ref, b1_ref,
                w2_ref, b2_ref, w3_ref, b3_ref, out_ref):
    h = jnp.maximum(
        jnp.dot(x8_ref[...], w1x_ref[...],
                preferred_element_type=jnp.float32)
        + jnp.dot(f8_ref[...], w1f_ref[...],
                  preferred_element_type=jnp.float32)
        + b1_ref[...], 0.0)
    h = jnp.maximum(
        jnp.dot(h, w2_ref[...], preferred_element_type=jnp.float32)
        + b2_ref[...], 0.0)
    out_ref[...] = (jnp.dot(h, w3_ref[...],
                            preferred_element_type=jnp.float32)
                    + b3_ref[...])


def _build_table(x, feature, w1, b1, w2, b2, w3, b3):
    x8 = x.reshape(12500, 48)
    f8 = feature.reshape(12500, 128)
    eye8 = jnp.eye(8, dtype=jnp.float32)
    w1x = jnp.kron(eye8, w1[:6])
    w1f = jnp.kron(eye8, w1[6:])
    w2_8 = jnp.kron(eye8, w2)
    w3_8 = jnp.kron(eye8, w3)
    b1_8 = jnp.tile(b1, 8).reshape(1, 128)
    b2_8 = jnp.tile(b2, 8).reshape(1, 128)
    b3_8 = jnp.tile(b3, 8).reshape(1, 128)
    e8 = pl.pallas_call(
        _table_body,
        out_shape=jax.ShapeDtypeStruct((12500, 128), jnp.float32),
    )(x8, f8, w1x, w1f, b1_8, w2_8, b2_8, w3_8, b3_8)
    flat = jnp.concatenate([e8.reshape(-1), f8.reshape(-1),
                            jnp.zeros(NZ * D, jnp.float32)])
    return flat.reshape(TROWS, D)


def _winner_body(f_hbm, part_hbm, wloc_v, chunk_v):
    cid = lax.axis_index("c")
    sid = lax.axis_index("s")
    w = sid * 2 + cid

    @pl.when(w < NP)
    def _():
        kbase = w * KSLICE

        def init_body(i, _):
            neg = jnp.full((16,), -1, jnp.int32)
            for u in range(8):
                wloc_v[pl.ds(i * 128 + u * 16, 16)] = neg
            return 0
        lax.fori_loop(0, NPAD // 128, init_body, 0)

        pltpu.sync_copy(f_hbm.at[pl.ds(kbase, KSLICE)], chunk_v)

        lane = lax.iota(jnp.int32, 16)
        nxt_idx = jnp.minimum(lane + 1, 15).reshape(16, 1)
        gdn = lax.GatherDimensionNumbers(offset_dims=(),
                                         collapsed_slice_dims=(0,),
                                         start_index_map=(0,))

        def vec_body(vi, _):
            f = chunk_v[pl.ds(vi * 16, 16)]
            kv = kbase + vi * 16 + lane
            key = f * 16 + lane
            ks, vs = plsc.sort_key_val(key, kv)
            fs = lax.shift_right_arithmetic(ks, 4)
            nxt = lax.gather(fs, nxt_idx, gdn, (1,),
                             mode=lax.GatherScatterMode.PROMISE_IN_BOUNDS)
            keep = (fs != nxt) | (lane == 15)
            plsc.store_scatter(wloc_v, [fs], vs, mask=keep)
            return 0

        lax.fori_loop(0, KSLICE // 16, vec_body, 0)
        pltpu.sync_copy(wloc_v, part_hbm.at[w])


def _resolve_body(a_hbm, b_hbm, t_hbm, part_hbm, out_hbm,
                  winner_v, mbuf_v, kidx_v, la_v, lb_v, rows_a_v, rows_b_v,
                  sem_a, sem_b):
    cid = lax.axis_index("c")
    sid = lax.axis_index("s")
    w = sid * 2 + cid
    lo = jnp.where(w == NW - 1, N - NODE_SPAN, w * NODE_STRIDE)

    for cc in range(NODE_SPAN // NCHUNK):
        nb = lo + cc * NCHUNK

        with jax.named_scope("ph_merge"):
            pltpu.sync_copy(part_hbm.at[0, pl.ds(nb, CSPAN)], winner_v)
            for j in range(1, NP):
                pltpu.sync_copy(part_hbm.at[j, pl.ds(nb, CSPAN)], mbuf_v)

                def merge_body(vi, _):
                    sl = pl.ds(vi * 16, 16)
                    winner_v[sl] = jnp.maximum(winner_v[sl], mbuf_v[sl])
                    return 0
                lax.fori_loop(0, CSPAN // 16, merge_body, 0)

        _ns_kidx = jax.named_scope("ph_kidx"); _ns_kidx.__enter__()

        def kidx_body(vi, _):
            wv = winner_v[pl.ds(vi * 16, 16)]
            row = vi // 8
            col = (vi % 8) * 16
            kidx_v[row, pl.ds(col, 16)] = jnp.maximum(wv, 0)
            return 0
        lax.fori_loop(0, IDXROWS * 8, kidx_body, 0)
        _ns_kidx.__exit__(None, None, None)

        with jax.named_scope("ph_idxgather"):
            descs = []
            for j in range(IDXROWS):
                descs.append(pltpu.async_copy(a_hbm.at[kidx_v.at[j]],
                                              la_v.at[j], sem_a))
                descs.append(pltpu.async_copy(b_hbm.at[kidx_v.at[j]],
                                              lb_v.at[j], sem_b))
            for d in descs:
                d.wait()

        _ns_fix = jax.named_scope("ph_fix"); _ns_fix.__enter__()

        def fix_body(vi, _):
            wv = winner_v[pl.ds(vi * 16, 16)]
            m = wv >= 0
            node = nb + vi * 16 + lax.iota(jnp.int32, 16)
            row = vi // 8
            col = pl.ds((vi % 8) * 16, 16)
            la = la_v[row, col]
            la_v[row, col] = jnp.where(m, la, N + node)
            lb = lb_v[row, col]
            lb_v[row, col] = jnp.where(m, lb, 2 * N + (node & (NZ - 1)))
            return 0
        lax.fori_loop(0, IDXROWS * 8, fix_body, 0)
        _ns_fix.__exit__(None, None, None)

        with jax.named_scope("ph_rowgather"):
            descs = []
            for j in range(IDXROWS):
                descs.append(pltpu.async_copy(t_hbm.at[la_v.at[j]],
                                              rows_a_v.at[pl.ds(j * 128, 128)],
                                              sem_a))
                descs.append(pltpu.async_copy(t_hbm.at[lb_v.at[j]],
                                              rows_b_v.at[pl.ds(j * 128, 128)],
                                              sem_b))
            for d in descs:
                d.wait()

        _ns_add = jax.named_scope("ph_add"); _ns_add.__enter__()

        def add_body(r, _):
            for u in range(4):
                rr = r * 4 + u
                rows_a_v[rr, :] = rows_a_v[rr, :] + rows_b_v[rr, :]
            return 0
        lax.fori_loop(0, NCHUNK // 4, add_body, 0)
        _ns_add.__exit__(None, None, None)

        with jax.named_scope("ph_out"):
            pltpu.sync_copy(rows_a_v.at[pl.ds(0, NCHUNK)],
                            out_hbm.at[pl.ds(nb, NCHUNK)])


def _make_sc_kernels():
    mesh = plsc.VectorSubcoreMesh(core_axis_name="c", subcore_axis_name="s",
                                  num_cores=2, num_subcores=16)
    winner_partials = pl.kernel(
        _winner_body,
        out_type=jax.ShapeDtypeStruct((NP, NPAD), jnp.int32),
        mesh=mesh,
        compiler_params=_SC_PARAMS,
        scratch_types=[
            pltpu.VMEM((NPAD,), jnp.int32),
            pltpu.VMEM((KSLICE,), jnp.int32),
        ],
    )
    resolve = pl.kernel(
        _resolve_body,
        out_type=jax.ShapeDtypeStruct((N, D), jnp.float32),
        mesh=mesh,
        compiler_params=_SC_PARAMS,
        scratch_types=[
            pltpu.VMEM((CSPAN,), jnp.int32),
            pltpu.VMEM((CSPAN,), jnp.int32),
            pltpu.VMEM((IDXROWS, 128), jnp.int32),
            pltpu.VMEM((IDXROWS, 128), jnp.int32),
            pltpu.VMEM((IDXROWS, 128), jnp.int32),
            pltpu.VMEM((CSPAN, D), jnp.float32),
            pltpu.VMEM((CSPAN, D), jnp.float32),
            pltpu.SemaphoreType.DMA,
            pltpu.SemaphoreType.DMA,
        ],
    )
    return winner_partials, resolve


def kernel(X, Feature, I_list, W1, b1, W2, b2, W3, b3):
    tri = I_list[:, 0, :, :]
    a_list = tri[..., 0].reshape(-1).astype(jnp.int32)
    b_list = tri[..., 1].reshape(-1).astype(jnp.int32)
    fathers = tri[..., 2].reshape(-1).astype(jnp.int32)
    winner_partials, resolve = _make_sc_kernels()
    table = _build_table(X, Feature, W1, b1, W2, b2, W3, b3)
    partials = winner_partials(fathers)
    return resolve(a_list, b_list, table, partials)


# speedup vs baseline: 1.4632x; 1.4632x over previous
import functools

import jax
import jax.numpy as jnp
from jax import lax
from jax.experimental import pallas as pl
from jax.experimental.pallas import tpu as pltpu
from jax.experimental.pallas import tpu_sc as plsc

N = 100000
NPAD = 100096
D = 16
NZ = 64
TROWS = 2 * N + NZ
NK = 131072

NW = 32
NP = 8
KSLICE = NK // NP
NODE_SPAN = 3200
NODE_STRIDE = 3120
NCHUNK = 1600
IDXROWS = 13
CSPAN = IDXROWS * 128

_SC_PARAMS = pltpu.CompilerParams(needs_layout_passes=False,
                                  use_tc_tiling_on_sc=False)


def _table_body(x8_ref, f8_ref, w1x_ref, w1f_ref, b1_ref,
                w2_ref, b2_ref, w3_ref, b3_ref, out_ref):
    h = jnp.maximum(
        jnp.dot(x8_ref[...], w1x_ref[...],
                preferred_element_type=jnp.float32)
        + jnp.dot(f8_ref[...], w1f_ref[...],
                  preferred_element_type=jnp.float32)
        + b1_ref[...], 0.0)
    h = jnp.maximum(
        jnp.dot(h, w2_ref[...], preferred_element_type=jnp.float32)
        + b2_ref[...], 0.0)
    out_ref[...] = (jnp.dot(h, w3_ref[...],
                            preferred_element_type=jnp.float32)
                    + b3_ref[...])


def _build_table(x, feature, w1, b1, w2, b2, w3, b3):
    x8 = x.reshape(12500, 48)
    f8 = feature.reshape(12500, 128)
    eye8 = jnp.eye(8, dtype=jnp.float32)
    w1x = jnp.kron(eye8, w1[:6])
    w1f = jnp.kron(eye8, w1[6:])
    w2_8 = jnp.kron(eye8, w2)
    w3_8 = jnp.kron(eye8, w3)
    b1_8 = jnp.tile(b1, 8).reshape(1, 128)
    b2_8 = jnp.tile(b2, 8).reshape(1, 128)
    b3_8 = jnp.tile(b3, 8).reshape(1, 128)
    e8 = pl.pallas_call(
        _table_body,
        out_shape=jax.ShapeDtypeStruct((12500, 128), jnp.float32),
    )(x8, f8, w1x, w1f, b1_8, w2_8, b2_8, w3_8, b3_8)
    flat = jnp.concatenate([e8.reshape(-1), f8.reshape(-1),
                            jnp.zeros(NZ * D, jnp.float32)])
    return flat.reshape(TROWS, D)


def _winner_body(f_hbm, part_hbm, wloc_v, chunk_v):
    cid = lax.axis_index("c")
    sid = lax.axis_index("s")
    w = sid * 2 + cid

    @pl.when(w < NP)
    def _():
        kbase = w * KSLICE
        lane = lax.iota(jnp.int32, 16)

        fall0 = lane - NPAD

        def init_body(i, _):
            base = i * 128
            for u in range(8):
                wloc_v[pl.ds(base + u * 16, 16)] = fall0 + (base + u * 16)
            return 0
        lax.fori_loop(0, NPAD // 128, init_body, 0)

        pltpu.sync_copy(f_hbm.at[pl.ds(kbase, KSLICE)], chunk_v)

        nxt_idx = jnp.minimum(lane + 1, 15).reshape(16, 1)
        gdn = lax.GatherDimensionNumbers(offset_dims=(),
                                         collapsed_slice_dims=(0,),
                                         start_index_map=(0,))

        def vec_body(vi, _):
            f = chunk_v[pl.ds(vi * 16, 16)]
            kv = kbase + vi * 16 + lane
            key = f * 16 + lane
            ks, vs = plsc.sort_key_val(key, kv)
            fs = lax.shift_right_arithmetic(ks, 4)
            nxt = lax.gather(fs, nxt_idx, gdn, (1,),
                             mode=lax.GatherScatterMode.PROMISE_IN_BOUNDS)
            keep = (fs != nxt) | (lane == 15)
            plsc.store_scatter(wloc_v, [fs], vs, mask=keep)
            return 0

        lax.fori_loop(0, KSLICE // 16, vec_body, 0)
        pltpu.sync_copy(wloc_v, part_hbm.at[w])


def _resolve_body(a_hbm, b_hbm, t_hbm, part_hbm, out_hbm,
                  winner_v, mbuf_v, kidx_v, la_v, lb_v, rows_a_v, rows_b_v,
                  sem_a, sem_b):
    cid = lax.axis_index("c")
    sid = lax.axis_index("s")
    w = sid * 2 + cid
    lo = jnp.where(w == NW - 1, N - NODE_SPAN, w * NODE_STRIDE)

    for cc in range(NODE_SPAN // NCHUNK):
        nb = lo + cc * NCHUNK

        pltpu.sync_copy(part_hbm.at[0, pl.ds(nb, CSPAN)], winner_v)
        for j in range(1, NP - 1):
            pltpu.sync_copy(part_hbm.at[j, pl.ds(nb, CSPAN)], mbuf_v)

            def merge_body(vi, _):
                for u in range(4):
                    sl = pl.ds((vi * 4 + u) * 16, 16)
                    winner_v[sl] = jnp.maximum(winner_v[sl], mbuf_v[sl])
                return 0
            lax.fori_loop(0, CSPAN // 64, merge_body, 0)

        pltpu.sync_copy(part_hbm.at[NP - 1, pl.ds(nb, CSPAN)], mbuf_v)

        def merge_last(vi, _):
            for u in range(8):
                col = u * 16
                sl = pl.ds(vi * 128 + col, 16)
                kidx_v[vi, pl.ds(col, 16)] = (
                    jnp.maximum(winner_v[sl], mbuf_v[sl]) + NPAD)
            return 0
        lax.fori_loop(0, IDXROWS, merge_last, 0)

        descs = []
        for j in range(IDXROWS):
            descs.append(pltpu.async_copy(a_hbm.at[kidx_v.at[j]],
                                          la_v.at[j], sem_a))
            descs.append(pltpu.async_copy(b_hbm.at[kidx_v.at[j]],
                                          lb_v.at[j], sem_b))
        for d in descs:
            d.wait()

        descs = []
        for j in range(IDXROWS):
            descs.append(pltpu.async_copy(t_hbm.at[la_v.at[j]],
                                          rows_a_v.at[pl.ds(j * 128, 128)],
                                          sem_a))
            descs.append(pltpu.async_copy(t_hbm.at[lb_v.at[j]],
                                          rows_b_v.at[pl.ds(j * 128, 128)],
                                          sem_b))
        for d in descs:
            d.wait()

        def add_body(r, _):
            for u in range(8):
                rr = r * 8 + u
                rows_a_v[rr, :] = rows_a_v[rr, :] + rows_b_v[rr, :]
            return 0
        lax.fori_loop(0, NCHUNK // 8, add_body, 0)

        pltpu.sync_copy(rows_a_v.at[pl.ds(0, NCHUNK)],
                        out_hbm.at[pl.ds(nb, NCHUNK)])


def _make_sc_kernels():
    mesh = plsc.VectorSubcoreMesh(core_axis_name="c", subcore_axis_name="s",
                                  num_cores=2, num_subcores=16)
    winner_partials = pl.kernel(
        _winner_body,
        out_type=jax.ShapeDtypeStruct((NP, NPAD), jnp.int32),
        mesh=mesh,
        compiler_params=_SC_PARAMS,
        scratch_types=[
            pltpu.VMEM((NPAD,), jnp.int32),
            pltpu.VMEM((KSLICE,), jnp.int32),
        ],
    )
    resolve = pl.kernel(
        _resolve_body,
        out_type=jax.ShapeDtypeStruct((N, D), jnp.float32),
        mesh=mesh,
        compiler_params=_SC_PARAMS,
        scratch_types=[
            pltpu.VMEM((CSPAN,), jnp.int32),
            pltpu.VMEM((CSPAN,), jnp.int32),
            pltpu.VMEM((IDXROWS, 128), jnp.int32),
            pltpu.VMEM((IDXROWS, 128), jnp.int32),
            pltpu.VMEM((IDXROWS, 128), jnp.int32),
            pltpu.VMEM((CSPAN, D), jnp.float32),
            pltpu.VMEM((CSPAN, D), jnp.float32),
            pltpu.SemaphoreType.DMA,
            pltpu.SemaphoreType.DMA,
        ],
    )
    return winner_partials, resolve


def kernel(X, Feature, I_list, W1, b1, W2, b2, W3, b3):
    tri = I_list[:, 0, :, :]
    a_list = tri[..., 0].reshape(-1).astype(jnp.int32)
    b_list = tri[..., 1].reshape(-1).astype(jnp.int32)
    fathers = tri[..., 2].reshape(-1).astype(jnp.int32)
    nn = jnp.arange(NPAD, dtype=jnp.int32)
    a_ext = jnp.concatenate([jnp.minimum(N + nn, TROWS - 1), a_list])
    b_ext = jnp.concatenate([2 * N + (nn & (NZ - 1)), b_list])
    winner_partials, resolve = _make_sc_kernels()
    table = _build_table(X, Feature, W1, b1, W2, b2, W3, b3)
    partials = winner_partials(fathers)
    return resolve(a_ext, b_ext, table, partials)


# speedup vs baseline: 1.6073x; 1.0985x over previous
import functools

import jax
import jax.numpy as jnp
from jax import lax
from jax.experimental import pallas as pl
from jax.experimental.pallas import tpu as pltpu
from jax.experimental.pallas import tpu_sc as plsc

N = 100000
NPAD = 100096
D = 16
NZ = 64
TROWS = 2 * N + NZ
NK = 131072

NW = 32
NP = 8
KSLICE = NK // NP
NODE_SPAN = 3200
NODE_STRIDE = 3120
NCHUNK = 1600
IDXROWS = 13
CSPAN = IDXROWS * 128

_SC_PARAMS = pltpu.CompilerParams(needs_layout_passes=False,
                                  use_tc_tiling_on_sc=False)


def _table_body(x8_ref, f8_ref, w1x_ref, w1f_ref, b1_ref,
                w2_ref, b2_ref, w3_ref, b3_ref, out_ref):
    h = jnp.maximum(
        jnp.dot(x8_ref[...], w1x_ref[...],
                preferred_element_type=jnp.float32)
        + jnp.dot(f8_ref[...], w1f_ref[...],
                  preferred_element_type=jnp.float32)
        + b1_ref[...], 0.0)
    h = jnp.maximum(
        jnp.dot(h, w2_ref[...], preferred_element_type=jnp.float32)
        + b2_ref[...], 0.0)
    out_ref[...] = (jnp.dot(h, w3_ref[...],
                            preferred_element_type=jnp.float32)
                    + b3_ref[...])


def _build_table(x, feature, w1, b1, w2, b2, w3, b3):
    x_flat = lax.optimization_barrier(x.reshape(-1))
    f_flat = lax.optimization_barrier(feature.reshape(-1))
    x8 = x_flat.reshape(12500, 48)
    f8 = f_flat.reshape(12500, 128)
    eye8 = jnp.eye(8, dtype=jnp.float32)
    w1x = jnp.kron(eye8, w1[:6])
    w1f = jnp.kron(eye8, w1[6:])
    w2_8 = jnp.kron(eye8, w2)
    w3_8 = jnp.kron(eye8, w3)
    b1_8 = jnp.tile(b1, 8).reshape(1, 128)
    b2_8 = jnp.tile(b2, 8).reshape(1, 128)
    b3_8 = jnp.tile(b3, 8).reshape(1, 128)
    e8 = pl.pallas_call(
        _table_body,
        out_shape=jax.ShapeDtypeStruct((12500, 128), jnp.float32),
    )(x8, f8, w1x, w1f, b1_8, w2_8, b2_8, w3_8, b3_8)
    flat = jnp.concatenate([e8.reshape(-1), f_flat,
                            jnp.zeros(NZ * D, jnp.float32)])
    return flat.reshape(TROWS, D)


def _winner_body(f_hbm, part_hbm, wloc_v, chunk_v):
    cid = lax.axis_index("c")
    sid = lax.axis_index("s")
    w = sid * 2 + cid

    @pl.when(w < NP)
    def _():
        kbase = w * KSLICE
        lane = lax.iota(jnp.int32, 16)

        fall0 = lane - NPAD

        def init_body(i, _):
            base = i * 128
            for u in range(8):
                wloc_v[pl.ds(base + u * 16, 16)] = fall0 + (base + u * 16)
            return 0
        lax.fori_loop(0, NPAD // 128, init_body, 0)

        pltpu.sync_copy(f_hbm.at[pl.ds(kbase, KSLICE)], chunk_v)

        nxt_idx = jnp.minimum(lane + 1, 15).reshape(16, 1)
        gdn = lax.GatherDimensionNumbers(offset_dims=(),
                                         collapsed_slice_dims=(0,),
                                         start_index_map=(0,))

        def vec_body(vi, _):
            f = chunk_v[pl.ds(vi * 16, 16)]
            kv = kbase + vi * 16 + lane
            key = f * 16 + lane
            ks, vs = plsc.sort_key_val(key, kv)
            fs = lax.shift_right_arithmetic(ks, 4)
            nxt = lax.gather(fs, nxt_idx, gdn, (1,),
                             mode=lax.GatherScatterMode.PROMISE_IN_BOUNDS)
            keep = (fs != nxt) | (lane == 15)
            plsc.store_scatter(wloc_v, [fs], vs, mask=keep)
            return 0

        lax.fori_loop(0, KSLICE // 16, vec_body, 0)
        pltpu.sync_copy(wloc_v, part_hbm.at[w])


def _resolve_body(a_hbm, b_hbm, t_hbm, part_hbm, out_hbm,
                  winner_v, mbuf_v, kidx_v, la_v, lb_v, rows_a_v, rows_b_v,
                  sem_a, sem_b):
    cid = lax.axis_index("c")
    sid = lax.axis_index("s")
    w = sid * 2 + cid
    lo = jnp.where(w == NW - 1, N - NODE_SPAN, w * NODE_STRIDE)

    for cc in range(NODE_SPAN // NCHUNK):
        nb = lo + cc * NCHUNK

        pltpu.sync_copy(part_hbm.at[0, pl.ds(nb, CSPAN)], winner_v)
        for j in range(1, NP - 1):
            pltpu.sync_copy(part_hbm.at[j, pl.ds(nb, CSPAN)], mbuf_v)

            def merge_body(vi, _):
                for u in range(4):
                    sl = pl.ds((vi * 4 + u) * 16, 16)
                    winner_v[sl] = jnp.maximum(winner_v[sl], mbuf_v[sl])
                return 0
            lax.fori_loop(0, CSPAN // 64, merge_body, 0)

        pltpu.sync_copy(part_hbm.at[NP - 1, pl.ds(nb, CSPAN)], mbuf_v)

        def merge_last(vi, _):
            for u in range(8):
                col = u * 16
                sl = pl.ds(vi * 128 + col, 16)
                kidx_v[vi, pl.ds(col, 16)] = (
                    jnp.maximum(winner_v[sl], mbuf_v[sl]) + NPAD)
            return 0
        lax.fori_loop(0, IDXROWS, merge_last, 0)

        descs = []
        for j in range(IDXROWS):
            descs.append(pltpu.async_copy(a_hbm.at[kidx_v.at[j]],
                                          la_v.at[j], sem_a))
            descs.append(pltpu.async_copy(b_hbm.at[kidx_v.at[j]],
                                          lb_v.at[j], sem_b))
        for d in descs:
            d.wait()

        descs = []
        for j in range(IDXROWS):
            descs.append(pltpu.async_copy(t_hbm.at[la_v.at[j]],
                                          rows_a_v.at[pl.ds(j * 128, 128)],
                                          sem_a))
            descs.append(pltpu.async_copy(t_hbm.at[lb_v.at[j]],
                                          rows_b_v.at[pl.ds(j * 128, 128)],
                                          sem_b))
        for d in descs:
            d.wait()

        def add_body(r, _):
            for u in range(8):
                rr = r * 8 + u
                rows_a_v[rr, :] = rows_a_v[rr, :] + rows_b_v[rr, :]
            return 0
        lax.fori_loop(0, NCHUNK // 8, add_body, 0)

        pltpu.sync_copy(rows_a_v.at[pl.ds(0, NCHUNK)],
                        out_hbm.at[pl.ds(nb, NCHUNK)])


def _make_sc_kernels():
    mesh = plsc.VectorSubcoreMesh(core_axis_name="c", subcore_axis_name="s",
                                  num_cores=2, num_subcores=16)
    winner_partials = pl.kernel(
        _winner_body,
        out_type=jax.ShapeDtypeStruct((NP, NPAD), jnp.int32),
        mesh=mesh,
        compiler_params=_SC_PARAMS,
        scratch_types=[
            pltpu.VMEM((NPAD,), jnp.int32),
            pltpu.VMEM((KSLICE,), jnp.int32),
        ],
    )
    resolve = pl.kernel(
        _resolve_body,
        out_type=jax.ShapeDtypeStruct((N, D), jnp.float32),
        mesh=mesh,
        compiler_params=_SC_PARAMS,
        scratch_types=[
            pltpu.VMEM((CSPAN,), jnp.int32),
            pltpu.VMEM((CSPAN,), jnp.int32),
            pltpu.VMEM((IDXROWS, 128), jnp.int32),
            pltpu.VMEM((IDXROWS, 128), jnp.int32),
            pltpu.VMEM((IDXROWS, 128), jnp.int32),
            pltpu.VMEM((CSPAN, D), jnp.float32),
            pltpu.VMEM((CSPAN, D), jnp.float32),
            pltpu.SemaphoreType.DMA,
            pltpu.SemaphoreType.DMA,
        ],
    )
    return winner_partials, resolve


def kernel(X, Feature, I_list, W1, b1, W2, b2, W3, b3):
    tri = I_list[:, 0, :, :]
    a_list = tri[..., 0].reshape(-1).astype(jnp.int32)
    b_list = tri[..., 1].reshape(-1).astype(jnp.int32)
    fathers = tri[..., 2].reshape(-1).astype(jnp.int32)
    nn = jnp.arange(NPAD, dtype=jnp.int32)
    a_ext = jnp.concatenate([jnp.minimum(N + nn, TROWS - 1), a_list])
    b_ext = jnp.concatenate([2 * N + (nn & (NZ - 1)), b_list])
    winner_partials, resolve = _make_sc_kernels()
    table = _build_table(X, Feature, W1, b1, W2, b2, W3, b3)
    partials = winner_partials(fathers)
    return resolve(a_ext, b_ext, table, partials)


# speedup vs baseline: 1.6392x; 1.0198x over previous
import functools

import jax
import jax.numpy as jnp
from jax import lax
from jax.experimental import pallas as pl
from jax.experimental.pallas import tpu as pltpu
from jax.experimental.pallas import tpu_sc as plsc

N = 100000
NPAD = 100096
D = 16
NZ = 64
TROWS = 2 * N + NZ
NK = 131072

NW = 32
NP = 8
KSLICE = NK // NP
NODE_SPAN = 3200
NODE_STRIDE = 3120
NCHUNK = 1600
IDXROWS = 13
CSPAN = IDXROWS * 128

_SC_PARAMS = pltpu.CompilerParams(needs_layout_passes=False,
                                  use_tc_tiling_on_sc=False)


def _table_body(x8_ref, fm_ref, w1x_ref, w1f_ref, b1_ref,
                w2_ref, b2_ref, w3_ref, b3_ref, out_ref):
    f8 = fm_ref[...].reshape(12500, 128)
    h = jnp.maximum(
        jnp.dot(x8_ref[...], w1x_ref[...],
                preferred_element_type=jnp.float32)
        + jnp.dot(f8, w1f_ref[...],
                  preferred_element_type=jnp.float32)
        + b1_ref[...], 0.0)
    h = jnp.maximum(
        jnp.dot(h, w2_ref[...], preferred_element_type=jnp.float32)
        + b2_ref[...], 0.0)
    e = (jnp.dot(h, w3_ref[...], preferred_element_type=jnp.float32)
         + b3_ref[...])
    out_ref[pl.ds(0, N * D)] = e.reshape(N * D)
    out_ref[pl.ds(N * D, N * D)] = fm_ref[...]
    out_ref[pl.ds(2 * N * D, NZ * D)] = jnp.zeros(NZ * D, jnp.float32)


def _build_table(x, feature, w1, b1, w2, b2, w3, b3):
    x_flat = lax.optimization_barrier(x.reshape(-1))
    f_flat = lax.optimization_barrier(feature.reshape(-1))
    x8 = x_flat.reshape(12500, 48)
    eye8 = jnp.eye(8, dtype=jnp.float32)
    w1x = jnp.kron(eye8, w1[:6])
    w1f = jnp.kron(eye8, w1[6:])
    w2_8 = jnp.kron(eye8, w2)
    w3_8 = jnp.kron(eye8, w3)
    b1_8 = jnp.tile(b1, 8).reshape(1, 128)
    b2_8 = jnp.tile(b2, 8).reshape(1, 128)
    b3_8 = jnp.tile(b3, 8).reshape(1, 128)
    flat = pl.pallas_call(
        _table_body,
        out_shape=jax.ShapeDtypeStruct((TROWS * D,), jnp.float32),
    )(x8, f_flat, w1x, w1f, b1_8, w2_8, b2_8, w3_8, b3_8)
    return flat.reshape(TROWS, D)


def _winner_body(f_hbm, part_hbm, wloc_v, chunk_v):
    cid = lax.axis_index("c")
    sid = lax.axis_index("s")
    w = sid * 2 + cid

    @pl.when(w < NP)
    def _():
        kbase = w * KSLICE
        lane = lax.iota(jnp.int32, 16)

        fall0 = lane - NPAD

        def init_body(i, _):
            base = i * 128
            for u in range(8):
                wloc_v[pl.ds(base + u * 16, 16)] = fall0 + (base + u * 16)
            return 0
        lax.fori_loop(0, NPAD // 128, init_body, 0)

        pltpu.sync_copy(f_hbm.at[pl.ds(kbase, KSLICE)], chunk_v)

        nxt_idx = jnp.minimum(lane + 1, 15).reshape(16, 1)
        gdn = lax.GatherDimensionNumbers(offset_dims=(),
                                         collapsed_slice_dims=(0,),
                                         start_index_map=(0,))

        def vec_body(vi, _):
            f = chunk_v[pl.ds(vi * 16, 16)]
            kv = kbase + vi * 16 + lane
            key = f * 16 + lane
            ks, vs = plsc.sort_key_val(key, kv)
            fs = lax.shift_right_arithmetic(ks, 4)
            nxt = lax.gather(fs, nxt_idx, gdn, (1,),
                             mode=lax.GatherScatterMode.PROMISE_IN_BOUNDS)
            keep = (fs != nxt) | (lane == 15)
            plsc.store_scatter(wloc_v, [fs], vs, mask=keep)
            return 0

        lax.fori_loop(0, KSLICE // 16, vec_body, 0)
        pltpu.sync_copy(wloc_v, part_hbm.at[w])


def _resolve_body(a_hbm, b_hbm, t_hbm, part_hbm, out_hbm,
                  winner_v, mbuf_v, kidx_v, la_v, lb_v, rows_a_v, rows_b_v,
                  sem_a, sem_b):
    cid = lax.axis_index("c")
    sid = lax.axis_index("s")
    w = sid * 2 + cid
    lo = jnp.where(w == NW - 1, N - NODE_SPAN, w * NODE_STRIDE)

    for cc in range(NODE_SPAN // NCHUNK):
        nb = lo + cc * NCHUNK

        pltpu.sync_copy(part_hbm.at[0, pl.ds(nb, CSPAN)], winner_v)
        for j in range(1, NP - 1):
            pltpu.sync_copy(part_hbm.at[j, pl.ds(nb, CSPAN)], mbuf_v)

            def merge_body(vi, _):
                for u in range(4):
                    sl = pl.ds((vi * 4 + u) * 16, 16)
                    winner_v[sl] = jnp.maximum(winner_v[sl], mbuf_v[sl])
                return 0
            lax.fori_loop(0, CSPAN // 64, merge_body, 0)

        pltpu.sync_copy(part_hbm.at[NP - 1, pl.ds(nb, CSPAN)], mbuf_v)

        def merge_last(vi, _):
            for u in range(8):
                col = u * 16
                sl = pl.ds(vi * 128 + col, 16)
                kidx_v[vi, pl.ds(col, 16)] = (
                    jnp.maximum(winner_v[sl], mbuf_v[sl]) + NPAD)
            return 0
        lax.fori_loop(0, IDXROWS, merge_last, 0)

        descs = []
        for j in range(IDXROWS):
            descs.append(pltpu.async_copy(a_hbm.at[kidx_v.at[j]],
                                          la_v.at[j], sem_a))
            descs.append(pltpu.async_copy(b_hbm.at[kidx_v.at[j]],
                                          lb_v.at[j], sem_b))
        for d in descs:
            d.wait()

        descs = []
        for j in range(IDXROWS):
            descs.append(pltpu.async_copy(t_hbm.at[la_v.at[j]],
                                          rows_a_v.at[pl.ds(j * 128, 128)],
                                          sem_a))
            descs.append(pltpu.async_copy(t_hbm.at[lb_v.at[j]],
                                          rows_b_v.at[pl.ds(j * 128, 128)],
                                          sem_b))
        for d in descs:
            d.wait()

        def add_body(r, _):
            for u in range(8):
                rr = r * 8 + u
                rows_a_v[rr, :] = rows_a_v[rr, :] + rows_b_v[rr, :]
            return 0
        lax.fori_loop(0, NCHUNK // 8, add_body, 0)

        pltpu.sync_copy(rows_a_v.at[pl.ds(0, NCHUNK)],
                        out_hbm.at[pl.ds(nb, NCHUNK)])


def _make_sc_kernels():
    mesh = plsc.VectorSubcoreMesh(core_axis_name="c", subcore_axis_name="s",
                                  num_cores=2, num_subcores=16)
    winner_partials = pl.kernel(
        _winner_body,
        out_type=jax.ShapeDtypeStruct((NP, NPAD), jnp.int32),
        mesh=mesh,
        compiler_params=_SC_PARAMS,
        scratch_types=[
            pltpu.VMEM((NPAD,), jnp.int32),
            pltpu.VMEM((KSLICE,), jnp.int32),
        ],
    )
    resolve = pl.kernel(
        _resolve_body,
        out_type=jax.ShapeDtypeStruct((N, D), jnp.float32),
        mesh=mesh,
        compiler_params=_SC_PARAMS,
        scratch_types=[
            pltpu.VMEM((CSPAN,), jnp.int32),
            pltpu.VMEM((CSPAN,), jnp.int32),
            pltpu.VMEM((IDXROWS, 128), jnp.int32),
            pltpu.VMEM((IDXROWS, 128), jnp.int32),
            pltpu.VMEM((IDXROWS, 128), jnp.int32),
            pltpu.VMEM((CSPAN, D), jnp.float32),
            pltpu.VMEM((CSPAN, D), jnp.float32),
            pltpu.SemaphoreType.DMA,
            pltpu.SemaphoreType.DMA,
        ],
    )
    return winner_partials, resolve


def kernel(X, Feature, I_list, W1, b1, W2, b2, W3, b3):
    tri = I_list[:, 0, :, :]
    a_list = tri[..., 0].reshape(-1).astype(jnp.int32)
    b_list = tri[..., 1].reshape(-1).astype(jnp.int32)
    fathers = tri[..., 2].reshape(-1).astype(jnp.int32)
    nn = jnp.arange(NPAD, dtype=jnp.int32)
    a_ext = jnp.concatenate([jnp.minimum(N + nn, TROWS - 1), a_list])
    b_ext = jnp.concatenate([2 * N + (nn & (NZ - 1)), b_list])
    winner_partials, resolve = _make_sc_kernels()
    table = _build_table(X, Feature, W1, b1, W2, b2, W3, b3)
    partials = winner_partials(fathers)
    out = resolve(a_ext, b_ext, table, partials)
    return lax.optimization_barrier(out.reshape(-1)).reshape(N, D)
